# Initial kernel scaffold; baseline (speedup 1.0000x reference)
import jax
import jax.numpy as jnp
from jax.experimental import pallas as pl


def kernel(x_edges, x_edges_values, x_nodes_coord, x_tour, x_best_tour, x_tour_directed, params):
    raise NotImplementedError("write your pallas kernel here")


# speedup vs baseline: 1.5094x; 1.5094x over previous
import functools
import numpy as np
import jax
import jax.numpy as jnp
from jax import lax
from jax.experimental import pallas as pl
from jax.experimental.pallas import tpu as pltpu
from jax.experimental.pallas import tpu_sc as plsc

B, V, H = 16, 100, 128
VV = V * V
NE = B * VV
RT = 20
NT = V // RT
BR = RT * V
F32 = jnp.float32

RTH = 50
NTH = V // RTH
BRH = RTH * V

_brow = np.arange(BR)
SEGT20_NP = (_brow[:, None] // V == np.arange(RT)[None, :]).astype(np.float32)
TILEL_NP = (_brow[:, None] % V == np.arange(V)[None, :]).astype(np.float32)
SEGL_NP = (np.arange(RT)[:, None] == _brow[None, :] // V).astype(np.float32)
_hrow = np.arange(BRH)
SEGTH_NP = (_hrow[:, None] // V == np.arange(RTH)[None, :]).astype(np.float32)
TILELH_NP = (_hrow[:, None] % V == np.arange(V)[None, :]).astype(np.float32)
SEGLH_NP = (np.arange(RTH)[:, None] == _hrow[None, :] // V).astype(np.float32)


def _mm(a, b):
    return jnp.dot(a, b, preferred_element_type=F32,
                   precision=lax.Precision.HIGHEST)


def _dgT(a, b, ca, cb):
    return lax.dot_general(a, b, (((ca,), (cb,)), ((), ())),
                           preferred_element_type=F32,
                           precision=lax.Precision.HIGHEST)


def _s0_body(coords_ref, wn_ref, bn_ref, v1_ref, v2_ref, bm_ref,
             x_ref, v1x_ref, v2x_ref, bx_ref):
    x = _mm(coords_ref[...], wn_ref[...]) + bn_ref[0:1, :]
    x_ref[...] = x
    v1x_ref[...] = _mm(x, v1_ref[...])
    v2x_ref[...] = _mm(x, v2_ref[...])
    bx_ref[...] = _mm(x, bm_ref[...])


def _sl_body(x_ref, gbx_ref, gsum_ref, a_ref, v1_ref, v2_ref, bm_ref,
             x_ref_o, v1x_ref, v2x_ref, bx_ref):
    x = x_ref[...]
    t = _mm(x, a_ref[...]) + gbx_ref[...] / (gsum_ref[...] + 1e-20)
    mu = jnp.mean(t, axis=0, keepdims=True)
    var = jnp.mean((t - mu) * (t - mu), axis=0, keepdims=True)
    xn = x + jnp.maximum((t - mu) * lax.rsqrt(var + 1e-5), 0.0)
    x_ref_o[...] = xn
    v1x_ref[...] = _mm(xn, v1_ref[...])
    v2x_ref[...] = _mm(xn, v2_ref[...])
    bx_ref[...] = _mm(xn, bm_ref[...])


def _edge_tail(e_loc, u_ref, v1x_ref, v2x_ref, bx_ref, segt20_ref, tilel_ref,
               segl_ref, e_ref, en_ref, stats_ref, gsum_ref, gbx_ref):
    en = _mm(e_loc, u_ref[...])
    en = en + _mm(segt20_ref[...], v1x_ref[0, 0])
    en = en + _mm(tilel_ref[...], v2x_ref[0])
    e_ref[0] = e_loc
    en_ref[0] = en
    gate = jax.nn.sigmoid(en)
    gsum_ref[0, 0] = _mm(segl_ref[...], gate)
    bxb = _mm(tilel_ref[...], bx_ref[0])
    gbx_ref[0, 0] = _mm(segl_ref[...], gate * bxb)

    first = (pl.program_id(0) == 0) & (pl.program_id(1) == 0)

    @pl.when(first)
    def _():
        stats_ref[...] = jnp.zeros((8, 128), F32)

    stats_ref[0:1, :] += jnp.sum(en, axis=0, keepdims=True)
    stats_ref[1:2, :] += jnp.sum(en * en, axis=0, keepdims=True)


def _b0_body(feat_ref, embp_ref, v1x_ref, v2x_ref, bx_ref,
             u_ref, segt20_ref, tilel_ref, segl_ref,
             e_ref, en_ref, stats_ref, gsum_ref, gbx_ref):
    e_loc = _mm(feat_ref[0], embp_ref[...])
    _edge_tail(e_loc, u_ref, v1x_ref, v2x_ref, bx_ref, segt20_ref, tilel_ref,
               segl_ref, e_ref, en_ref, stats_ref, gsum_ref, gbx_ref)


def _bl_body(ep_ref, enp_ref, statsp_ref, v1x_ref, v2x_ref, bx_ref,
             u_ref, segt20_ref, tilel_ref, segl_ref,
             e_ref, en_ref, stats_ref, gsum_ref, gbx_ref):
    mu = statsp_ref[0:1, :] / NE
    var = statsp_ref[1:2, :] / NE - mu * mu
    rstd = lax.rsqrt(var + 1e-5)
    e_loc = ep_ref[0] + jnp.maximum((enp_ref[0] - mu) * rstd, 0.0)
    _edge_tail(e_loc, u_ref, v1x_ref, v2x_ref, bx_ref, segt20_ref, tilel_ref,
               segl_ref, e_ref, en_ref, stats_ref, gsum_ref, gbx_ref)


def _h1_body(ep_ref, enp_ref, statsp_ref, wn2_ref, bpr_ref, e3_ref, x2_ref):
    mu = statsp_ref[0:1, :] / NE
    var = statsp_ref[1:2, :] / NE - mu * mu
    rstd = lax.rsqrt(var + 1e-5)
    e3 = ep_ref[0] + jnp.maximum((enp_ref[0] - mu) * rstd, 0.0)
    e3_ref[0] = e3
    x2_ref[0] = _mm(e3, wn2_ref[...]) + bpr_ref[...]


SC_NC, SC_NS = 2, 16
SC_NW = SC_NC * SC_NS
SC_RPW = NE // SC_NW
SC_CH = 40
SC_NCH = SC_RPW // SC_CH


TS_PAD = 2048
TS_CH = TS_PAD // SC_NW


def _sc_gather_body(x2_hbm, e3_hbm, gidx_hbm, tsidx_hbm, out_hbm, ts_hbm,
                    idx_v, rows_v, idx2_v, rows2_v, sem):
    c = lax.axis_index("c")
    s = lax.axis_index("s")
    wid = s * SC_NC + c
    base = wid * SC_RPW

    def step(i, _):
        off = base + i * SC_CH
        pltpu.sync_copy(gidx_hbm.at[pl.ds(off, SC_CH)], idx_v)
        pltpu.async_copy(x2_hbm.at[idx_v], rows_v, sem).wait()
        pltpu.sync_copy(rows_v, out_hbm.at[pl.ds(off, SC_CH)])
        return 0

    lax.fori_loop(0, SC_NCH, step, 0)

    off2 = wid * TS_CH
    pltpu.sync_copy(tsidx_hbm.at[pl.ds(off2, TS_CH)], idx2_v)
    pltpu.async_copy(e3_hbm.at[idx2_v], rows2_v, sem).wait()
    pltpu.sync_copy(rows2_v, ts_hbm.at[pl.ds(off2, TS_CH)])


def _sc_gather(x2_flat, e3_flat, gidx, tsidx):
    run = functools.partial(
        pl.kernel,
        out_type=(jax.ShapeDtypeStruct((NE, H), F32),
                  jax.ShapeDtypeStruct((TS_PAD, H), F32)),
        mesh=plsc.VectorSubcoreMesh(core_axis_name="c", subcore_axis_name="s"),
        scratch_types=[
            pltpu.VMEM((SC_CH,), jnp.int32),
            pltpu.VMEM((SC_CH, H), F32),
            pltpu.VMEM((TS_CH,), jnp.int32),
            pltpu.VMEM((TS_CH, H), F32),
            pltpu.SemaphoreType.DMA,
        ],
    )(_sc_gather_body)
    return run(x2_flat, e3_flat, gidx, tsidx)


def _h2_body(e3_ref, n2_ref, ts_ref, xd_ref, vals_ref, gpq_ref,
             segt20_ref, tilel_ref, segl_ref, wsplit_ref, smallp_ref, w3_ref,
             aw1_ref, aw2_ref, acc_ref):
    it = pl.program_id(1)
    Ps = xd_ref[0]
    segt20 = segt20_ref[...]
    tilel = tilel_ref[...]

    io_r = lax.broadcasted_iota(jnp.int32, (V, V), 0).astype(F32)
    io_c = lax.broadcasted_iota(jnp.int32, (V, V), 1).astype(F32)
    eye = jnp.where(io_r == io_c, 1.0, 0.0)

    io_tr = lax.broadcasted_iota(jnp.int32, (RTH, V), 0)
    io_tc = lax.broadcasted_iota(jnp.int32, (RTH, V), 1)
    sel = jnp.where(io_tr + it * RTH == io_tc, 1.0, 0.0)

    succ_col = jnp.sum(Ps * io_c, axis=1, keepdims=True)
    s_col = io_r[:, 0:1]
    u_col = jnp.minimum(s_col, succ_col)
    v_col = jnp.maximum(s_col, succ_col)
    key_col = u_col * V + v_col
    key_row = _dgT(key_col, eye, 0, 0)
    r_col = jnp.sum(jnp.where(key_row < key_col, 1.0, 0.0),
                    axis=1, keepdims=True)
    d_col = jnp.sum(vals_ref[0] * Ps, axis=1, keepdims=True)

    vals2 = _dgT(_mm(Ps, vals_ref[0]), Ps, 1, 1)
    Pr = jnp.where(r_col == io_c, 1.0, 0.0)
    g_st = _dgT(_mm(Pr, gpq_ref[0]), Pr, 1, 1)

    vals_t = _mm(sel, vals_ref[0])
    vals2_t = _mm(sel, vals2)
    g_t = _mm(sel, g_st)
    r_s = _mm(sel, r_col)
    r_row = _dgT(r_col, eye, 0, 0)
    d_s = _mm(sel, d_col)
    d_row = _dgT(d_col, eye, 0, 0)
    succ_s = _mm(sel, succ_col)
    succ_row = _dgT(succ_col, eye, 0, 0)

    cost2d = vals_t + vals2_t - d_s - d_row

    ones_col = jnp.ones((V, 1), F32)
    costf = _mm(_mm(segt20, cost2d) * tilel, ones_col)

    ts_full = ts_ref[0]
    a1f = _mm(segt20, _mm(_mm(sel, ts_full), wsplit_ref[0:H, :]))
    a2f = _mm(tilel, _mm(ts_full, wsplit_ref[H:2 * H, :]))
    x1f = _mm(e3_ref[0], wsplit_ref[2 * H:3 * H, :])

    wc = smallp_ref[0:1, :]
    h = x1f + n2_ref[0] + a1f + a2f + costf * wc
    h = jnp.maximum(h + smallp_ref[2:3, :], 0.0)
    h = jnp.maximum(_mm(h, aw1_ref[...]) + smallp_ref[3:4, :], 0.0)
    h = jnp.maximum(_mm(h, aw2_ref[...]) + smallp_ref[4:5, :], 0.0)
    logits_col = _mm(h, w3_ref[...]) + smallp_ref[6:7, 0:1]

    L = _mm(segl_ref[...], logits_col * tilel)

    valid = r_s < r_row
    kmat = r_s * V - r_s * (r_s + 1.0) * 0.5 + (r_row - r_s - 1.0)

    neg = jnp.float32(-3e38)
    total = jnp.where(valid, L + g_t, neg)
    tm = jnp.max(total)
    tk = jnp.min(jnp.where((total == tm) & valid, kmat, jnp.float32(3e38)))
    selm = jnp.where((kmat == tk) & valid, 1.0, 0.0)

    s2d = (io_tr + it * RTH).astype(F32)
    t2d = io_tc.astype(F32)
    s_star_t = jnp.sum(selm * s2d)
    t_star_t = jnp.sum(selm * t2d)
    ss_t = jnp.sum(selm * succ_s)
    st_t = jnp.sum(selm * succ_row)
    lstar_t = jnp.sum(selm * L)

    t_lmax = jnp.max(jnp.where(valid, L, neg))
    t_ssum = jnp.sum(jnp.where(valid, jnp.exp(L - t_lmax), 0.0))

    @pl.when(it == 0)
    def _():
        rows0 = lax.broadcasted_iota(jnp.int32, (16, 128), 0)
        init = jnp.where(rows0 == 0, neg, 0.0)
        init = init + jnp.where(rows0 == 1, jnp.float32(3e38), 0.0)
        init = init + jnp.where(rows0 == 7, neg, 0.0)
        acc_ref[0] = init

    acc = acc_ref[0]
    am = acc[0:1, 0:1]
    ak = acc[1:2, 0:1]
    a_s = acc[2:3, 0:1]
    a_t = acc[3:4, 0:1]
    a_ss = acc[4:5, 0:1]
    a_st = acc[5:6, 0:1]
    a_ls = acc[6:7, 0:1]
    a_lm = acc[7:8, 0:1]
    a_sum = acc[8:9, 0:1]

    better = (tm > am) | ((tm == am) & (tk < ak))
    nm = jnp.where(better, tm, am)
    nk = jnp.where(better, tk, ak)
    ns = jnp.where(better, s_star_t, a_s)
    nt = jnp.where(better, t_star_t, a_t)
    nss = jnp.where(better, ss_t, a_ss)
    nst = jnp.where(better, st_t, a_st)
    nls = jnp.where(better, lstar_t, a_ls)

    nlm = jnp.maximum(a_lm, t_lmax)
    nsum = a_sum * jnp.exp(a_lm - nlm) + t_ssum * jnp.exp(t_lmax - nlm)
    pi = nls - (nlm + jnp.log(nsum))

    rows = lax.broadcasted_iota(jnp.int32, (16, 128), 0)
    out = jnp.where(rows == 0, nm, 0.0)
    out = out + jnp.where(rows == 1, nk, 0.0)
    out = out + jnp.where(rows == 2, ns, 0.0)
    out = out + jnp.where(rows == 3, nt, 0.0)
    out = out + jnp.where(rows == 4, nss, 0.0)
    out = out + jnp.where(rows == 5, nst, 0.0)
    out = out + jnp.where(rows == 6, nls, 0.0)
    out = out + jnp.where(rows == 7, nlm, 0.0)
    out = out + jnp.where(rows == 8, nsum, 0.0)
    out = out + jnp.where(rows == 9, pi, 0.0)
    acc_ref[0] = out


def kernel(x_edges, x_edges_values, x_nodes_coord, x_tour, x_best_tour,
           x_tour_directed, params):
    p = params
    vals = x_edges_values
    valsf = vals.reshape(B, VV, 1)
    xtf = x_tour.reshape(B, VV, 1)
    xbf = x_best_tour.reshape(B, VV, 1)

    segt20 = jnp.asarray(SEGT20_NP)
    tilel = jnp.asarray(TILEL_NP)
    segl = jnp.asarray(SEGL_NP)

    z64 = jnp.zeros((64,), F32)
    embp = jnp.stack([
        jnp.concatenate([p['W_eval'][0], z64]),
        jnp.concatenate([z64, p['W_ecat'][0]]),
        jnp.concatenate([z64, p['W_ecat'][1]]),
        jnp.concatenate([p['b_eval'], p['b_ecat']]),
    ] + [jnp.zeros((128,), F32)] * 4)

    coords_pad = jnp.concatenate(
        [x_nodes_coord.reshape(B * V, 2), jnp.zeros((B * V, 6), F32)], axis=1)
    wn_pad = jnp.concatenate([p['W_node'], jnp.zeros((6, H), F32)], axis=0)

    mm2 = jax.ShapeDtypeStruct((B * V, H), F32)
    x0, v1x, v2x, bx = pl.pallas_call(
        _s0_body,
        out_shape=(mm2, mm2, mm2, mm2),
    )(coords_pad, wn_pad, p['b_node'].reshape(1, H),
      p['layers'][0]['V1'], p['layers'][0]['V2'], p['layers'][0]['Bm'])

    e_shape = jax.ShapeDtypeStruct((B, VV, H), F32)
    stats_shape = jax.ShapeDtypeStruct((8, 128), F32)
    agg_shape = jax.ShapeDtypeStruct((B, NT, RT, H), F32)

    big_in_common = [
        pl.BlockSpec((1, 1, RT, H), lambda b, i: (b, i, 0, 0)),
        pl.BlockSpec((1, V, H), lambda b, i: (b, 0, 0)),
        pl.BlockSpec((1, V, H), lambda b, i: (b, 0, 0)),
        pl.BlockSpec((H, H), lambda b, i: (0, 0)),
        pl.BlockSpec((BR, RT), lambda b, i: (0, 0)),
        pl.BlockSpec((BR, V), lambda b, i: (0, 0)),
        pl.BlockSpec((RT, BR), lambda b, i: (0, 0)),
    ]
    big_out = [
        pl.BlockSpec((1, BR, H), lambda b, i: (b, i, 0)),
        pl.BlockSpec((1, BR, H), lambda b, i: (b, i, 0)),
        pl.BlockSpec((8, 128), lambda b, i: (0, 0)),
        pl.BlockSpec((1, 1, RT, H), lambda b, i: (b, i, 0, 0)),
        pl.BlockSpec((1, 1, RT, H), lambda b, i: (b, i, 0, 0)),
    ]
    big_out_shape = (e_shape, e_shape, stats_shape, agg_shape, agg_shape)

    feat = jnp.concatenate(
        [valsf, xtf, xbf, jnp.ones_like(valsf),
         jnp.zeros((B, VV, 4), F32)], axis=-1)
    e, en, stats, gsum, gbx = pl.pallas_call(
        _b0_body,
        grid=(B, NT),
        in_specs=[pl.BlockSpec((1, BR, 8), lambda b, i: (b, i, 0)),
                  pl.BlockSpec((8, 128), lambda b, i: (0, 0))] + big_in_common,
        out_specs=big_out,
        out_shape=big_out_shape,
    )(feat, embp,
      v1x.reshape(B, NT, RT, H), v2x.reshape(B, V, H), bx.reshape(B, V, H),
      p['layers'][0]['U'], segt20, tilel, segl)

    x_cur = x0
    for l in (1, 2):
        lp = p['layers'][l]
        x_cur, v1x, v2x, bx = pl.pallas_call(
            _sl_body,
            out_shape=(mm2, mm2, mm2, mm2),
        )(x_cur, gbx.reshape(B * V, H), gsum.reshape(B * V, H),
          p['layers'][l - 1]['A'], lp['V1'], lp['V2'], lp['Bm'])

        eblk = pl.BlockSpec((1, BR, H), lambda b, i: (b, i, 0))
        e, en, stats, gsum, gbx = pl.pallas_call(
            _bl_body,
            grid=(B, NT),
            in_specs=[eblk, eblk,
                      pl.BlockSpec((8, 128), lambda b, i: (0, 0))] + big_in_common,
            out_specs=big_out,
            out_shape=big_out_shape,
        )(e, en, stats,
          v1x.reshape(B, NT, RT, H), v2x.reshape(B, V, H), bx.reshape(B, V, H),
          lp['U'], segt20, tilel, segl)

    w_c = p['pre_W'][4 * H:5 * H, :]
    aw0 = p['act_W0']
    bpr = _mm(p['pre_b'] + _mm(p['cost_b'], w_c), aw0).reshape(1, H)
    eblk1 = pl.BlockSpec((1, BRH, H), lambda b, i: (b, i, 0))
    e3, x2 = pl.pallas_call(
        _h1_body,
        grid=(B, NTH),
        in_specs=[eblk1, eblk1, pl.BlockSpec((8, 128), lambda b, i: (0, 0)),
                  pl.BlockSpec((H, H), lambda b, i: (0, 0)),
                  pl.BlockSpec((1, H), lambda b, i: (0, 0))],
        out_specs=[eblk1, eblk1],
        out_shape=(e_shape, e_shape),
    )(e, en, stats, _mm(p['pre_W'][3 * H:4 * H, :], aw0), bpr)

    succ = jnp.argmax(x_tour_directed, axis=-1).astype(jnp.int32)
    boff = (jnp.arange(B, dtype=jnp.int32) * VV)[:, None]
    gidx = (succ[:, :, None] * V + succ[:, None, :]
            + boff[:, :, None]).reshape(NE)
    ar_i = jnp.arange(V, dtype=jnp.int32)
    tsidx = (ar_i[None, :] * V + succ + boff).reshape(B * V)
    tsidx = jnp.concatenate(
        [tsidx, jnp.zeros((TS_PAD - B * V,), jnp.int32)])
    n2, ts_rows = _sc_gather(x2.reshape(NE, H), e3.reshape(NE, H), gidx, tsidx)
    n2 = n2.reshape(B, VV, H)
    ts = ts_rows[:B * V].reshape(B, V, H)

    nep = V * (V - 1) // 2
    g = jax.random.gumbel(jax.random.key(42), (B, nep), F32)
    rs, cs = jnp.triu_indices(V, k=1)
    gpq = jnp.zeros((B, V, V), F32).at[:, rs, cs].set(g)

    smallp = jnp.stack([
        _mm(_mm(p['cost_W'][0], w_c), aw0),
        jnp.zeros((128,), F32),
        p['act_b0'], p['act_b1'], p['act_b2'],
        jnp.zeros((128,), F32),
        jnp.full((128,), p['act_b3'][0], F32),
        jnp.zeros((128,), F32),
    ])

    vblk = pl.BlockSpec((1, V, V), lambda b, i: (b, 0, 0))
    eblk2 = pl.BlockSpec((1, BRH, H), lambda b, i: (b, i, 0))
    res = pl.pallas_call(
        _h2_body,
        grid=(B, NTH),
        in_specs=[eblk2, eblk2,
                  pl.BlockSpec((1, V, H), lambda b, i: (b, 0, 0)),
                  vblk, vblk, vblk,
                  pl.BlockSpec((BRH, RTH), lambda b, i: (0, 0)),
                  pl.BlockSpec((BRH, V), lambda b, i: (0, 0)),
                  pl.BlockSpec((RTH, BRH), lambda b, i: (0, 0)),
                  pl.BlockSpec((3 * H, H), lambda b, i: (0, 0)),
                  pl.BlockSpec((8, 128), lambda b, i: (0, 0)),
                  pl.BlockSpec((H, 1), lambda b, i: (0, 0)),
                  pl.BlockSpec((H, H), lambda b, i: (0, 0)),
                  pl.BlockSpec((H, H), lambda b, i: (0, 0))],
        out_specs=[pl.BlockSpec((1, 16, 128), lambda b, i: (b, 0, 0))],
        out_shape=(jax.ShapeDtypeStruct((B, 16, 128), F32),),
    )(e3, n2, ts, x_tour_directed, vals, gpq,
      jnp.asarray(SEGTH_NP), jnp.asarray(TILELH_NP), jnp.asarray(SEGLH_NP),
      _mm(p['pre_W'][0:3 * H, :], aw0), smallp,
      p['act_W3'], p['act_W1'], p['act_W2'])[0]

    r0 = res[:, :, 0]
    s_star = r0[:, 2]
    t_star = r0[:, 3]
    ss = r0[:, 4]
    st = r0[:, 5]
    actions = r0[:, 1].astype(jnp.int32)
    pi = r0[:, 9]
    barange = jnp.arange(B, dtype=jnp.int32)
    e1 = jnp.stack([barange, jnp.minimum(s_star, ss).astype(jnp.int32),
                    jnp.maximum(s_star, ss).astype(jnp.int32)], axis=1)
    e2 = jnp.stack([barange, jnp.minimum(t_star, st).astype(jnp.int32),
                    jnp.maximum(t_star, st).astype(jnp.int32)], axis=1)
    edges = jnp.stack([e1, e2], axis=1)
    return edges, pi, actions
